# Initial kernel scaffold; baseline (speedup 1.0000x reference)
import jax
import jax.numpy as jnp
from jax.experimental import pallas as pl


def kernel(x, edge_index, W_l0, b_l0, W_r0, gamma0, beta0, W_l1, b_l1, W_r1, gamma1, beta1, W_l2, b_l2, W_r2, gamma2, beta2, W_proj, W_cls, b_cls):
    raise NotImplementedError("write your pallas kernel here")


# speedup vs baseline: 5.3853x; 5.3853x over previous
import functools

import jax
import jax.numpy as jnp
from jax import lax
from jax.experimental import pallas as pl
from jax.experimental.pallas import tpu as pltpu
from jax.experimental.pallas import tpu_sc as plsc

_N = 10000
_E = 320000
_H = 128
_C = 2
_EPS = 1e-5

_NC = 2
_NS = 16
_NW = _NC * _NS
_L = 16

_W = _H + 16
_K = 128
_CPW = -(-_E // (_NW * _K))
_EPW = _CPW * _K
_EPAD = _NW * _EPW
_NACC = 10240
_ZR = _NACC // _NS
_PADROWS = _NACC - _N

_mesh = plsc.VectorSubcoreMesh(core_axis_name="c", subcore_axis_name="s")


@functools.partial(
    pl.kernel,
    out_type=jax.ShapeDtypeStruct((_NC * _NACC, _W), jnp.float32),
    mesh=_mesh,
    scratch_types=[
        pltpu.VMEM((_K,), jnp.int32),
        pltpu.VMEM((_K,), jnp.int32),
        pltpu.VMEM((_K, _W), jnp.float32),
        pltpu.VMEM_SHARED((_NACC, _W), jnp.float32),
        pltpu.SemaphoreType.DMA,
    ],
    compiler_params=pltpu.CompilerParams(use_tc_tiling_on_sc=False),
)
def _seg_sum(h_hbm, src_hbm, dst_hbm, out_hbm, src_v, dst_v, rows_v,
             acc_sh, sem):
  c = lax.axis_index("c")
  s = lax.axis_index("s")
  w = s * _NC + c

  def zrow(i, carry):
    for j in range(_W // _L):
      rows_v[i, pl.ds(j * _L, _L)] = jnp.zeros((_L,), jnp.float32)
    return carry
  lax.fori_loop(0, _K, zrow, 0)
  for k in range(_ZR // _K):
    pltpu.sync_copy(rows_v, acc_sh.at[pl.ds(s * _ZR + k * _K, _K)])
  plsc.subcore_barrier()

  def chunk(g, carry):
    base = (w * _CPW + g) * _K
    pltpu.sync_copy(src_hbm.at[pl.ds(base, _K)], src_v)
    pltpu.sync_copy(dst_hbm.at[pl.ds(base, _K)], dst_v)
    pltpu.async_copy(h_hbm.at[src_v], rows_v, sem).wait()
    pltpu.sync_copy(rows_v, acc_sh.at[dst_v], add=True)
    return carry
  lax.fori_loop(0, _CPW, chunk, 0)
  plsc.subcore_barrier()

  for k in range(_ZR // _K):
    pltpu.sync_copy(acc_sh.at[pl.ds(s * _ZR + k * _K, _K)], rows_v)
    pltpu.sync_copy(rows_v, out_hbm.at[pl.ds(c * _NACC + s * _ZR + k * _K, _K)])


def _tc_layer(partials_ref, h_ref, wl_ref, bl_ref, wr_ref, g_ref, b_ref,
              p_ref, h_out):
  p = partials_ref[...]
  psum = p[0:_NACC, :] + p[_NACC:2 * _NACC, :]
  inv = 1.0 / jnp.maximum(psum[: _N, _H:_H + 1], 1.0)
  mean = psum[: _N, 0:_H] * inv
  h_in = h_ref[:, 0:_H]
  t = (jnp.dot(mean, wl_ref[...], preferred_element_type=jnp.float32)
       + bl_ref[...]
       + jnp.dot(h_in, wr_ref[...], preferred_element_type=jnp.float32))
  mu = jnp.mean(t, axis=0, keepdims=True)
  d = t - mu
  var = jnp.mean(d * d, axis=0, keepdims=True)
  h = d * lax.rsqrt(var + _EPS) * g_ref[...] + b_ref[...]
  h = jnp.maximum(h, 0.0)
  h = h + jnp.dot(h_in, p_ref[...], preferred_element_type=jnp.float32)
  h_out[:, 0:_H] = h
  h_out[:, _H:_H + 1] = jnp.ones((_N, 1), jnp.float32)
  h_out[:, _H + 1:_W] = jnp.zeros((_N, _W - _H - 1), jnp.float32)


def _tc_cls(h_ref, wcls_ref, bcls_ref, out_ref):
  h = h_ref[:, 0:_H]
  out_ref[...] = (jnp.dot(h, wcls_ref[...], preferred_element_type=jnp.float32)
                  + bcls_ref[...])


def kernel(x, edge_index, W_l0, b_l0, W_r0, gamma0, beta0, W_l1, b_l1, W_r1,
           gamma1, beta1, W_l2, b_l2, W_r2, gamma2, beta2, W_proj, W_cls,
           b_cls):
  src = edge_index[0]
  dst = edge_index[1]
  npad = _EPAD - _E
  pad_ids = jnp.arange(npad, dtype=jnp.int32)
  src_pad = jnp.concatenate([src, pad_ids % _N])
  dst_pad = jnp.concatenate([dst, _N + pad_ids % _PADROWS])

  x_ext = jnp.concatenate(
      [x, jnp.ones((_N, 1), jnp.float32), jnp.zeros((_N, 15), jnp.float32)],
      axis=1)

  eye = jnp.eye(_H, dtype=jnp.float32)
  wl_s = jnp.stack([W_l0, W_l1, W_l2])
  bl_s = jnp.stack([b_l0, b_l1, b_l2])
  wr_s = jnp.stack([W_r0, W_r1, W_r2])
  g_s = jnp.stack([gamma0, gamma1, gamma2])
  b_s = jnp.stack([beta0, beta1, beta2])
  p_s = jnp.stack([W_proj, eye, eye])

  layer_call = pl.pallas_call(
      _tc_layer, out_shape=jax.ShapeDtypeStruct((_N, _W), jnp.float32))

  def body(i, h_ext):
    partials = _seg_sum(h_ext, src_pad, dst_pad)
    h_ext = layer_call(partials, h_ext, wl_s[i], bl_s[i], wr_s[i], g_s[i],
                       b_s[i], p_s[i])
    return h_ext

  h_ext = lax.fori_loop(0, 3, body, x_ext)
  out = pl.pallas_call(
      _tc_cls, out_shape=jax.ShapeDtypeStruct((_N, _C), jnp.float32),
  )(h_ext, W_cls, b_cls)
  return out


# speedup vs baseline: 8.0174x; 1.4887x over previous
import functools

import jax
import jax.numpy as jnp
from jax import lax
from jax.experimental import pallas as pl
from jax.experimental.pallas import tpu as pltpu
from jax.experimental.pallas import tpu_sc as plsc

_N = 10000
_E = 320000
_H = 128
_C = 2
_EPS = 1e-5

_NC = 2
_NS = 16
_NW = _NC * _NS
_L = 16

_W = _H + 16
_K = 128
_G = 4
_BODY = 2 * _G
_NBODY = 10
_CPW = _BODY * _NBODY
_EPW = _CPW * _K
_EPAD = _NW * _EPW
_IROWS = _EPAD // _K
_NACC = 10112
_ZR = _NACC // _NS
_PADROWS = _NACC - _N

_mesh = plsc.VectorSubcoreMesh(core_axis_name="c", subcore_axis_name="s")


@functools.partial(
    pl.kernel,
    out_type=jax.ShapeDtypeStruct((_NC * _NACC, _W), jnp.float32),
    mesh=_mesh,
    scratch_types=[
        pltpu.VMEM((_G, _K), jnp.int32),
        pltpu.VMEM((_G, _K), jnp.int32),
        pltpu.VMEM((_G, _K), jnp.int32),
        pltpu.VMEM((_G, _K), jnp.int32),
        pltpu.VMEM((_K, _W), jnp.float32),
        pltpu.VMEM((_K, _W), jnp.float32),
        pltpu.VMEM_SHARED((_NACC, _W), jnp.float32),
        pltpu.SemaphoreType.DMA,
        pltpu.SemaphoreType.DMA,
        pltpu.SemaphoreType.DMA,
        pltpu.SemaphoreType.DMA,
    ],
    compiler_params=pltpu.CompilerParams(use_tc_tiling_on_sc=False),
)
def _seg_sum(h_hbm, src_hbm, dst_hbm, out_hbm, srca, dsta, srcb, dstb,
             rows0, rows1, acc_sh, semg0, semg1, semia, semib):
  c = lax.axis_index("c")
  s = lax.axis_index("s")
  w = s * _NC + c
  rows = (rows0, rows1)
  semg = (semg0, semg1)

  def zrow(i, carry):
    for j in range(_W // _L):
      rows0[i, pl.ds(j * _L, _L)] = jnp.zeros((_L,), jnp.float32)
    return carry
  lax.fori_loop(0, _K, zrow, 0)
  zsizes = [_K] * (_ZR // _K) + ([_ZR % _K] if _ZR % _K else [])
  off = 0
  for sz in zsizes:
    pltpu.sync_copy(rows0.at[pl.ds(0, sz)], acc_sh.at[pl.ds(s * _ZR + off, sz)])
    off += sz
  plsc.subcore_barrier()

  row0 = w * _CPW
  pltpu.async_copy(src_hbm.at[pl.ds(row0, _G)], srca, semia)
  pltpu.async_copy(dst_hbm.at[pl.ds(row0, _G)], dsta, semia)

  def body(m, carry):
    base = row0 + m * _BODY
    pltpu.make_async_copy(src_hbm.at[pl.ds(base, _G)], srca, semia).wait()
    pltpu.make_async_copy(dst_hbm.at[pl.ds(base, _G)], dsta, semia).wait()
    pltpu.async_copy(src_hbm.at[pl.ds(base + _G, _G)], srcb, semib)
    pltpu.async_copy(dst_hbm.at[pl.ds(base + _G, _G)], dstb, semib)

    descs = [None] * _BODY
    descs[0] = pltpu.async_copy(h_hbm.at[srca.at[0]], rows0, semg0)
    for j in range(_BODY):
      sl = j % 2
      descs[j].wait()
      if j + 1 < _BODY:
        if j + 1 == _G:
          pltpu.make_async_copy(
              src_hbm.at[pl.ds(base + _G, _G)], srcb, semib).wait()
          pltpu.make_async_copy(
              dst_hbm.at[pl.ds(base + _G, _G)], dstb, semib).wait()
        nsrc = srca if j + 1 < _G else srcb
        descs[j + 1] = pltpu.async_copy(
            h_hbm.at[nsrc.at[(j + 1) % _G]], rows[1 - sl], semg[1 - sl])
      didx = dsta if j < _G else dstb
      pltpu.sync_copy(rows[sl], acc_sh.at[didx.at[j % _G]], add=True)
      if j == _BODY - 1:
        nbase = lax.rem(base + _BODY, _IROWS)
        pltpu.async_copy(src_hbm.at[pl.ds(nbase, _G)], srca, semia)
        pltpu.async_copy(dst_hbm.at[pl.ds(nbase, _G)], dsta, semia)
    return carry

  lax.fori_loop(0, _NBODY, body, 0)
  pltpu.make_async_copy(src_hbm.at[pl.ds(row0, _G)], srca, semia).wait()
  pltpu.make_async_copy(dst_hbm.at[pl.ds(row0, _G)], dsta, semia).wait()
  plsc.subcore_barrier()

  off = 0
  for sz in zsizes:
    pltpu.sync_copy(acc_sh.at[pl.ds(s * _ZR + off, sz)],
                    rows0.at[pl.ds(0, sz)])
    pltpu.sync_copy(rows0.at[pl.ds(0, sz)],
                    out_hbm.at[pl.ds(c * _NACC + s * _ZR + off, sz)])
    off += sz


def _tc_layer(partials_ref, h_ref, wl_ref, bl_ref, wr_ref, g_ref, b_ref,
              p_ref, h_out):
  p = partials_ref[...]
  psum = p[0:_NACC, :] + p[_NACC:2 * _NACC, :]
  inv = 1.0 / jnp.maximum(psum[: _N, _H:_H + 1], 1.0)
  mean = psum[: _N, 0:_H] * inv
  h_in = h_ref[:, 0:_H]
  t = (jnp.dot(mean, wl_ref[...], preferred_element_type=jnp.float32)
       + bl_ref[...]
       + jnp.dot(h_in, wr_ref[...], preferred_element_type=jnp.float32))
  mu = jnp.mean(t, axis=0, keepdims=True)
  d = t - mu
  var = jnp.mean(d * d, axis=0, keepdims=True)
  h = d * lax.rsqrt(var + _EPS) * g_ref[...] + b_ref[...]
  h = jnp.maximum(h, 0.0)
  h = h + jnp.dot(h_in, p_ref[...], preferred_element_type=jnp.float32)
  h_out[:, 0:_H] = h
  h_out[:, _H:_H + 1] = jnp.ones((_N, 1), jnp.float32)
  h_out[:, _H + 1:_W] = jnp.zeros((_N, _W - _H - 1), jnp.float32)


def _tc_cls(h_ref, wcls_ref, bcls_ref, out_ref):
  h = h_ref[:, 0:_H]
  out_ref[...] = (jnp.dot(h, wcls_ref[...], preferred_element_type=jnp.float32)
                  + bcls_ref[...])


def kernel(x, edge_index, W_l0, b_l0, W_r0, gamma0, beta0, W_l1, b_l1, W_r1,
           gamma1, beta1, W_l2, b_l2, W_r2, gamma2, beta2, W_proj, W_cls,
           b_cls):
  src = edge_index[0]
  dst = edge_index[1]
  npad = _EPAD - _E
  pad_ids = jnp.arange(npad, dtype=jnp.int32)
  src_pad = jnp.concatenate([src, pad_ids % _N]).reshape(_IROWS, _K)
  dst_pad = jnp.concatenate([dst, _N + pad_ids % _PADROWS]).reshape(_IROWS, _K)

  x_ext = jnp.concatenate(
      [x, jnp.ones((_N, 1), jnp.float32), jnp.zeros((_N, 15), jnp.float32)],
      axis=1)

  eye = jnp.eye(_H, dtype=jnp.float32)
  wl_s = jnp.stack([W_l0, W_l1, W_l2])
  bl_s = jnp.stack([b_l0, b_l1, b_l2])
  wr_s = jnp.stack([W_r0, W_r1, W_r2])
  g_s = jnp.stack([gamma0, gamma1, gamma2])
  b_s = jnp.stack([beta0, beta1, beta2])
  p_s = jnp.stack([W_proj, eye, eye])

  layer_call = pl.pallas_call(
      _tc_layer, out_shape=jax.ShapeDtypeStruct((_N, _W), jnp.float32))

  def body(i, h_ext):
    partials = _seg_sum(h_ext, src_pad, dst_pad)
    h_ext = layer_call(partials, h_ext, wl_s[i], bl_s[i], wr_s[i], g_s[i],
                       b_s[i], p_s[i])
    return h_ext

  h_ext = lax.fori_loop(0, 3, body, x_ext)
  out = pl.pallas_call(
      _tc_cls, out_shape=jax.ShapeDtypeStruct((_N, _C), jnp.float32),
  )(h_ext, W_cls, b_cls)
  return out


# speedup vs baseline: 8.2274x; 1.0262x over previous
import functools

import jax
import jax.numpy as jnp
from jax import lax
from jax.experimental import pallas as pl
from jax.experimental.pallas import tpu as pltpu
from jax.experimental.pallas import tpu_sc as plsc

_N = 10000
_E = 320000
_H = 128
_C = 2
_EPS = 1e-5

_NC = 2
_NS = 16
_NW = _NC * _NS
_L = 16

_W = _H + 16
_K = 128
_G = 4
_BODY = 2 * _G
_NBODY = 10
_CPW = _BODY * _NBODY
_EPW = _CPW * _K
_EPAD = _NW * _EPW
_IROWS = _EPAD // _K
_NACC = 10112
_ZR = _NACC // _NS
_PADROWS = _NACC - _N

_mesh = plsc.VectorSubcoreMesh(core_axis_name="c", subcore_axis_name="s")


@functools.partial(
    pl.kernel,
    out_type=jax.ShapeDtypeStruct((_NC * _NACC, _W), jnp.float32),
    mesh=_mesh,
    scratch_types=[
        pltpu.VMEM((_G, _K), jnp.int32),
        pltpu.VMEM((_G, _K), jnp.int32),
        pltpu.VMEM((_G, _K), jnp.int32),
        pltpu.VMEM((_G, _K), jnp.int32),
        pltpu.VMEM((_K, _W), jnp.float32),
        pltpu.VMEM((_K, _W), jnp.float32),
        pltpu.VMEM_SHARED((_NACC, _W), jnp.float32),
        pltpu.SemaphoreType.DMA,
        pltpu.SemaphoreType.DMA,
        pltpu.SemaphoreType.DMA,
        pltpu.SemaphoreType.DMA,
        pltpu.SemaphoreType.DMA,
        pltpu.SemaphoreType.DMA,
    ],
    compiler_params=pltpu.CompilerParams(use_tc_tiling_on_sc=False),
)
def _seg_sum(h_hbm, src_hbm, dst_hbm, out_hbm, srca, dsta, srcb, dstb,
             rows0, rows1, acc_sh, semg0, semg1, sems0, sems1, semia, semib):
  c = lax.axis_index("c")
  s = lax.axis_index("s")
  w = s * _NC + c
  rows = (rows0, rows1)
  semg = (semg0, semg1)
  sems = (sems0, sems1)

  def zrow(i, carry):
    for j in range(_W // _L):
      rows0[i, pl.ds(j * _L, _L)] = jnp.zeros((_L,), jnp.float32)
    return carry
  lax.fori_loop(0, _K, zrow, 0)
  zsizes = [_K] * (_ZR // _K) + ([_ZR % _K] if _ZR % _K else [])
  off = 0
  for sz in zsizes:
    pltpu.sync_copy(rows0.at[pl.ds(0, sz)], acc_sh.at[pl.ds(s * _ZR + off, sz)])
    off += sz
  plsc.subcore_barrier()

  row0 = w * _CPW
  pltpu.async_copy(src_hbm.at[pl.ds(row0, _G)], srca, semia)
  pltpu.async_copy(dst_hbm.at[pl.ds(row0, _G)], dsta, semia)

  def body(m, carry):
    base = row0 + m * _BODY
    pltpu.make_async_copy(src_hbm.at[pl.ds(base, _G)], srca, semia).wait()
    pltpu.make_async_copy(dst_hbm.at[pl.ds(base, _G)], dsta, semia).wait()
    pltpu.async_copy(src_hbm.at[pl.ds(base + _G, _G)], srcb, semib)
    pltpu.async_copy(dst_hbm.at[pl.ds(base + _G, _G)], dstb, semib)

    descs_g = [None] * _BODY
    descs_s = [None] * _BODY
    descs_g[0] = pltpu.async_copy(h_hbm.at[srca.at[0]], rows0, semg0)
    for j in range(_BODY):
      sl = j % 2
      descs_g[j].wait()
      didx = dsta if j < _G else dstb
      descs_s[j] = pltpu.async_copy(
          rows[sl], acc_sh.at[didx.at[j % _G]], sems[sl], add=True)
      if j + 1 < _BODY:
        if j + 1 == _G:
          pltpu.make_async_copy(
              src_hbm.at[pl.ds(base + _G, _G)], srcb, semib).wait()
          pltpu.make_async_copy(
              dst_hbm.at[pl.ds(base + _G, _G)], dstb, semib).wait()
        if j >= 1:
          descs_s[j - 1].wait()
        nsrc = srca if j + 1 < _G else srcb
        descs_g[j + 1] = pltpu.async_copy(
            h_hbm.at[nsrc.at[(j + 1) % _G]], rows[1 - sl], semg[1 - sl])
      if j == _BODY - 1:
        nbase = lax.rem(base + _BODY, _IROWS)
        pltpu.async_copy(src_hbm.at[pl.ds(nbase, _G)], srca, semia)
        pltpu.async_copy(dst_hbm.at[pl.ds(nbase, _G)], dsta, semia)
    descs_s[_BODY - 2].wait()
    descs_s[_BODY - 1].wait()
    return carry

  lax.fori_loop(0, _NBODY, body, 0)
  pltpu.make_async_copy(src_hbm.at[pl.ds(row0, _G)], srca, semia).wait()
  pltpu.make_async_copy(dst_hbm.at[pl.ds(row0, _G)], dsta, semia).wait()
  plsc.subcore_barrier()

  off = 0
  for sz in zsizes:
    pltpu.sync_copy(acc_sh.at[pl.ds(s * _ZR + off, sz)],
                    rows0.at[pl.ds(0, sz)])
    pltpu.sync_copy(rows0.at[pl.ds(0, sz)],
                    out_hbm.at[pl.ds(c * _NACC + s * _ZR + off, sz)])
    off += sz


def _tc_layer(partials_ref, h_ref, wl_ref, bl_ref, wr_ref, g_ref, b_ref,
              p_ref, h_out):
  p = partials_ref[...]
  psum = p[0:_NACC, :] + p[_NACC:2 * _NACC, :]
  inv = 1.0 / jnp.maximum(psum[: _N, _H:_H + 1], 1.0)
  mean = psum[: _N, 0:_H] * inv
  h_in = h_ref[:, 0:_H]
  t = (jnp.dot(mean, wl_ref[...], preferred_element_type=jnp.float32)
       + bl_ref[...]
       + jnp.dot(h_in, wr_ref[...], preferred_element_type=jnp.float32))
  mu = jnp.mean(t, axis=0, keepdims=True)
  d = t - mu
  var = jnp.mean(d * d, axis=0, keepdims=True)
  h = d * lax.rsqrt(var + _EPS) * g_ref[...] + b_ref[...]
  h = jnp.maximum(h, 0.0)
  h = h + jnp.dot(h_in, p_ref[...], preferred_element_type=jnp.float32)
  h_out[:, 0:_H] = h
  h_out[:, _H:_H + 1] = jnp.ones((_N, 1), jnp.float32)
  h_out[:, _H + 1:_W] = jnp.zeros((_N, _W - _H - 1), jnp.float32)


def _tc_cls(h_ref, wcls_ref, bcls_ref, out_ref):
  h = h_ref[:, 0:_H]
  out_ref[...] = (jnp.dot(h, wcls_ref[...], preferred_element_type=jnp.float32)
                  + bcls_ref[...])


def kernel(x, edge_index, W_l0, b_l0, W_r0, gamma0, beta0, W_l1, b_l1, W_r1,
           gamma1, beta1, W_l2, b_l2, W_r2, gamma2, beta2, W_proj, W_cls,
           b_cls):
  src = edge_index[0]
  dst = edge_index[1]
  npad = _EPAD - _E
  pad_ids = jnp.arange(npad, dtype=jnp.int32)
  src_pad = jnp.concatenate([src, pad_ids % _N]).reshape(_IROWS, _K)
  dst_pad = jnp.concatenate([dst, _N + pad_ids % _PADROWS]).reshape(_IROWS, _K)

  x_ext = jnp.concatenate(
      [x, jnp.ones((_N, 1), jnp.float32), jnp.zeros((_N, 15), jnp.float32)],
      axis=1)

  eye = jnp.eye(_H, dtype=jnp.float32)
  wl_s = jnp.stack([W_l0, W_l1, W_l2])
  bl_s = jnp.stack([b_l0, b_l1, b_l2])
  wr_s = jnp.stack([W_r0, W_r1, W_r2])
  g_s = jnp.stack([gamma0, gamma1, gamma2])
  b_s = jnp.stack([beta0, beta1, beta2])
  p_s = jnp.stack([W_proj, eye, eye])

  layer_call = pl.pallas_call(
      _tc_layer, out_shape=jax.ShapeDtypeStruct((_N, _W), jnp.float32))

  def body(i, h_ext):
    partials = _seg_sum(h_ext, src_pad, dst_pad)
    h_ext = layer_call(partials, h_ext, wl_s[i], bl_s[i], wr_s[i], g_s[i],
                       b_s[i], p_s[i])
    return h_ext

  h_ext = lax.fori_loop(0, 3, body, x_ext)
  out = pl.pallas_call(
      _tc_cls, out_shape=jax.ShapeDtypeStruct((_N, _C), jnp.float32),
  )(h_ext, W_cls, b_cls)
  return out


# speedup vs baseline: 8.3574x; 1.0158x over previous
import functools

import jax
import jax.numpy as jnp
from jax import lax
from jax.experimental import pallas as pl
from jax.experimental.pallas import tpu as pltpu
from jax.experimental.pallas import tpu_sc as plsc

_N = 10000
_E = 320000
_H = 128
_C = 2
_EPS = 1e-5

_NC = 2
_NS = 16
_NW = _NC * _NS
_L = 16

_W = _H + 16
_K = 128
_G = 8
_BODY = 2 * _G
_NBODY = 5
_CPW = _BODY * _NBODY
_EPW = _CPW * _K
_EPAD = _NW * _EPW
_IROWS = _EPAD // _K
_NACC = _N
_ZR = _NACC // _NS
_ZROWS = 128
_NROWS = _N + _ZROWS

_mesh = plsc.VectorSubcoreMesh(core_axis_name="c", subcore_axis_name="s")


@functools.partial(
    pl.kernel,
    out_type=jax.ShapeDtypeStruct((_NC * _NACC, _W), jnp.float32),
    mesh=_mesh,
    scratch_types=[
        pltpu.VMEM((_G, _K), jnp.int32),
        pltpu.VMEM((_G, _K), jnp.int32),
        pltpu.VMEM((_G, _K), jnp.int32),
        pltpu.VMEM((_G, _K), jnp.int32),
        pltpu.VMEM((_K, _W), jnp.float32),
        pltpu.VMEM((_K, _W), jnp.float32),
        pltpu.VMEM_SHARED((_NACC, _W), jnp.float32),
        pltpu.SemaphoreType.DMA,
        pltpu.SemaphoreType.DMA,
        pltpu.SemaphoreType.DMA,
        pltpu.SemaphoreType.DMA,
        pltpu.SemaphoreType.DMA,
        pltpu.SemaphoreType.DMA,
    ],
    compiler_params=pltpu.CompilerParams(use_tc_tiling_on_sc=False),
)
def _seg_sum(h_hbm, src_hbm, dst_hbm, out_hbm, srca, dsta, srcb, dstb,
             rows0, rows1, acc_sh, semg0, semg1, sems0, sems1, semia, semib):
  c = lax.axis_index("c")
  s = lax.axis_index("s")
  w = s * _NC + c
  rows = (rows0, rows1)
  semg = (semg0, semg1)
  sems = (sems0, sems1)

  row0 = w * _CPW
  pltpu.async_copy(src_hbm.at[pl.ds(row0, _G)], srca, semia)
  pltpu.async_copy(dst_hbm.at[pl.ds(row0, _G)], dsta, semia)

  def zrow(i, carry):
    for j in range(_W // _L):
      rows0[i, pl.ds(j * _L, _L)] = jnp.zeros((_L,), jnp.float32)
    return carry
  lax.fori_loop(0, _K, zrow, 0)
  zsizes = [_K] * (_ZR // _K) + ([_ZR % _K] if _ZR % _K else [])
  off = 0
  for sz in zsizes:
    pltpu.sync_copy(rows0.at[pl.ds(0, sz)], acc_sh.at[pl.ds(s * _ZR + off, sz)])
    off += sz
  plsc.subcore_barrier()


  def body(m, carry):
    base = row0 + m * _BODY
    pltpu.make_async_copy(src_hbm.at[pl.ds(base, _G)], srca, semia).wait()
    pltpu.make_async_copy(dst_hbm.at[pl.ds(base, _G)], dsta, semia).wait()
    pltpu.async_copy(src_hbm.at[pl.ds(base + _G, _G)], srcb, semib)
    pltpu.async_copy(dst_hbm.at[pl.ds(base + _G, _G)], dstb, semib)

    descs_g = [None] * _BODY
    descs_s = [None] * _BODY
    descs_g[0] = pltpu.async_copy(h_hbm.at[srca.at[0]], rows0, semg0)
    for j in range(_BODY):
      sl = j % 2
      descs_g[j].wait()
      didx = dsta if j < _G else dstb
      descs_s[j] = pltpu.async_copy(
          rows[sl], acc_sh.at[didx.at[j % _G]], sems[sl], add=True)
      if j + 1 < _BODY:
        if j + 1 == _G:
          pltpu.make_async_copy(
              src_hbm.at[pl.ds(base + _G, _G)], srcb, semib).wait()
          pltpu.make_async_copy(
              dst_hbm.at[pl.ds(base + _G, _G)], dstb, semib).wait()
        if j >= 1:
          descs_s[j - 1].wait()
        nsrc = srca if j + 1 < _G else srcb
        descs_g[j + 1] = pltpu.async_copy(
            h_hbm.at[nsrc.at[(j + 1) % _G]], rows[1 - sl], semg[1 - sl])
      if j == _BODY - 1:
        nbase = lax.rem(base + _BODY, _IROWS)
        pltpu.async_copy(src_hbm.at[pl.ds(nbase, _G)], srca, semia)
        pltpu.async_copy(dst_hbm.at[pl.ds(nbase, _G)], dsta, semia)
    descs_s[_BODY - 2].wait()
    descs_s[_BODY - 1].wait()
    return carry

  lax.fori_loop(0, _NBODY, body, 0)
  pltpu.make_async_copy(src_hbm.at[pl.ds(row0, _G)], srca, semia).wait()
  pltpu.make_async_copy(dst_hbm.at[pl.ds(row0, _G)], dsta, semia).wait()
  plsc.subcore_barrier()

  off = 0
  for sz in zsizes:
    pltpu.sync_copy(acc_sh.at[pl.ds(s * _ZR + off, sz)],
                    rows0.at[pl.ds(0, sz)])
    pltpu.sync_copy(rows0.at[pl.ds(0, sz)],
                    out_hbm.at[pl.ds(c * _NACC + s * _ZR + off, sz)])
    off += sz


def _tc_layer(partials_ref, h_ref, wl_ref, bl_ref, wr_ref, g_ref, b_ref,
              p_ref, h_out):
  p = partials_ref[...]
  psum = p[0:_NACC, :] + p[_NACC:2 * _NACC, :]
  inv = 1.0 / jnp.maximum(psum[:, _H:_H + 1], 1.0)
  mean = psum[:, 0:_H] * inv
  h_in = h_ref[0:_N, 0:_H]
  t = (jnp.dot(mean, wl_ref[...], preferred_element_type=jnp.float32)
       + bl_ref[...]
       + jnp.dot(h_in, wr_ref[...], preferred_element_type=jnp.float32))
  mu = jnp.mean(t, axis=0, keepdims=True)
  d = t - mu
  var = jnp.mean(d * d, axis=0, keepdims=True)
  h = d * lax.rsqrt(var + _EPS) * g_ref[...] + b_ref[...]
  h = jnp.maximum(h, 0.0)
  h = h + jnp.dot(h_in, p_ref[...], preferred_element_type=jnp.float32)
  h_out[0:_N, 0:_H] = h
  h_out[0:_N, _H:_H + 1] = jnp.ones((_N, 1), jnp.float32)
  h_out[0:_N, _H + 1:_W] = jnp.zeros((_N, _W - _H - 1), jnp.float32)
  h_out[_N:_NROWS, :] = jnp.zeros((_ZROWS, _W), jnp.float32)


def _tc_cls(h_ref, wcls_ref, bcls_ref, out_ref):
  h = h_ref[0:_N, 0:_H]
  out_ref[...] = (jnp.dot(h, wcls_ref[...], preferred_element_type=jnp.float32)
                  + bcls_ref[...])


def kernel(x, edge_index, W_l0, b_l0, W_r0, gamma0, beta0, W_l1, b_l1, W_r1,
           gamma1, beta1, W_l2, b_l2, W_r2, gamma2, beta2, W_proj, W_cls,
           b_cls):
  src = edge_index[0]
  dst = edge_index[1]
  npad = _EPAD - _E
  pad_ids = jnp.arange(npad, dtype=jnp.int32)
  src_pad = jnp.concatenate([src, _N + pad_ids % _ZROWS]).reshape(_IROWS, _K)
  dst_pad = jnp.concatenate([dst, pad_ids % _N]).reshape(_IROWS, _K)

  x_ext = jnp.concatenate(
      [x, jnp.ones((_N, 1), jnp.float32), jnp.zeros((_N, 15), jnp.float32)],
      axis=1)
  x_ext = jnp.concatenate(
      [x_ext, jnp.zeros((_ZROWS, _W), jnp.float32)], axis=0)

  eye = jnp.eye(_H, dtype=jnp.float32)
  wl_s = jnp.stack([W_l0, W_l1, W_l2])
  bl_s = jnp.stack([b_l0, b_l1, b_l2])
  wr_s = jnp.stack([W_r0, W_r1, W_r2])
  g_s = jnp.stack([gamma0, gamma1, gamma2])
  b_s = jnp.stack([beta0, beta1, beta2])
  p_s = jnp.stack([W_proj, eye, eye])

  layer_call = pl.pallas_call(
      _tc_layer, out_shape=jax.ShapeDtypeStruct((_NROWS, _W), jnp.float32))

  def body(i, h_ext):
    partials = _seg_sum(h_ext, src_pad, dst_pad)
    h_ext = layer_call(partials, h_ext, wl_s[i], bl_s[i], wr_s[i], g_s[i],
                       b_s[i], p_s[i])
    return h_ext

  h_ext = lax.fori_loop(0, 3, body, x_ext)
  out = pl.pallas_call(
      _tc_cls, out_shape=jax.ShapeDtypeStruct((_N, _C), jnp.float32),
  )(h_ext, W_cls, b_cls)
  return out


# speedup vs baseline: 10.4426x; 1.2495x over previous
import functools

import jax
import jax.numpy as jnp
from jax import lax
from jax.experimental import pallas as pl
from jax.experimental.pallas import tpu as pltpu
from jax.experimental.pallas import tpu_sc as plsc

_N = 10000
_E = 320000
_H = 128
_C = 2
_EPS = 1e-5

_NC = 2
_NS = 16
_NW = _NC * _NS
_L = 16

_W0 = _H + 16
_K = 128
_G = 8
_BODY = 2 * _G
_NBODY = 5
_CPW = _BODY * _NBODY
_EPW = _CPW * _K
_EPAD = _NW * _EPW
_IROWS = _EPAD // _K
_ZR = _N // _NS
_ZROWS = 128
_NROWS = _N + _ZROWS

_mesh = plsc.VectorSubcoreMesh(core_axis_name="c", subcore_axis_name="s")


def _make_seg_sum(width):
  @functools.partial(
      pl.kernel,
      out_type=jax.ShapeDtypeStruct((_NC * _N, width), jnp.float32),
      mesh=_mesh,
      scratch_types=[
          pltpu.VMEM((_G, _K), jnp.int32),
          pltpu.VMEM((_G, _K), jnp.int32),
          pltpu.VMEM((_G, _K), jnp.int32),
          pltpu.VMEM((_G, _K), jnp.int32),
          pltpu.VMEM((_K, width), jnp.float32),
          pltpu.VMEM((_K, width), jnp.float32),
          pltpu.VMEM_SHARED((_N, width), jnp.float32),
          pltpu.SemaphoreType.DMA,
          pltpu.SemaphoreType.DMA,
          pltpu.SemaphoreType.DMA,
          pltpu.SemaphoreType.DMA,
          pltpu.SemaphoreType.DMA,
          pltpu.SemaphoreType.DMA,
      ],
      compiler_params=pltpu.CompilerParams(use_tc_tiling_on_sc=False),
  )
  def seg(h_hbm, src_hbm, dst_hbm, out_hbm, srca, dsta, srcb, dstb,
          rows0, rows1, acc_sh, semg0, semg1, sems0, sems1, semia, semib):
    c = lax.axis_index("c")
    s = lax.axis_index("s")
    w = s * _NC + c
    rows = (rows0, rows1)
    semg = (semg0, semg1)
    sems = (sems0, sems1)

    row0 = w * _CPW
    pltpu.async_copy(src_hbm.at[pl.ds(row0, _G)], srca, semia)
    pltpu.async_copy(dst_hbm.at[pl.ds(row0, _G)], dsta, semia)

    def zrow(i, carry):
      for j in range(width // _L):
        rows0[i, pl.ds(j * _L, _L)] = jnp.zeros((_L,), jnp.float32)
      return carry
    lax.fori_loop(0, _K, zrow, 0)
    zsizes = [_K] * (_ZR // _K) + ([_ZR % _K] if _ZR % _K else [])
    off = 0
    for sz in zsizes:
      pltpu.sync_copy(rows0.at[pl.ds(0, sz)],
                      acc_sh.at[pl.ds(s * _ZR + off, sz)])
      off += sz
    plsc.subcore_barrier()

    def body(m, carry):
      base = row0 + m * _BODY
      pltpu.make_async_copy(src_hbm.at[pl.ds(base, _G)], srca, semia).wait()
      pltpu.make_async_copy(dst_hbm.at[pl.ds(base, _G)], dsta, semia).wait()
      pltpu.async_copy(src_hbm.at[pl.ds(base + _G, _G)], srcb, semib)
      pltpu.async_copy(dst_hbm.at[pl.ds(base + _G, _G)], dstb, semib)

      descs_g = [None] * _BODY
      descs_s = [None] * _BODY
      descs_g[0] = pltpu.async_copy(h_hbm.at[srca.at[0]], rows0, semg0)
      for j in range(_BODY):
        sl = j % 2
        descs_g[j].wait()
        didx = dsta if j < _G else dstb
        descs_s[j] = pltpu.async_copy(
            rows[sl], acc_sh.at[didx.at[j % _G]], sems[sl], add=True)
        if j + 1 < _BODY:
          if j + 1 == _G:
            pltpu.make_async_copy(
                src_hbm.at[pl.ds(base + _G, _G)], srcb, semib).wait()
            pltpu.make_async_copy(
                dst_hbm.at[pl.ds(base + _G, _G)], dstb, semib).wait()
          if j >= 1:
            descs_s[j - 1].wait()
          nsrc = srca if j + 1 < _G else srcb
          descs_g[j + 1] = pltpu.async_copy(
              h_hbm.at[nsrc.at[(j + 1) % _G]], rows[1 - sl], semg[1 - sl])
        if j == _BODY - 1:
          nbase = lax.rem(base + _BODY, _IROWS)
          pltpu.async_copy(src_hbm.at[pl.ds(nbase, _G)], srca, semia)
          pltpu.async_copy(dst_hbm.at[pl.ds(nbase, _G)], dsta, semia)
      descs_s[_BODY - 2].wait()
      descs_s[_BODY - 1].wait()
      return carry

    lax.fori_loop(0, _NBODY, body, 0)
    pltpu.make_async_copy(src_hbm.at[pl.ds(row0, _G)], srca, semia).wait()
    pltpu.make_async_copy(dst_hbm.at[pl.ds(row0, _G)], dsta, semia).wait()
    plsc.subcore_barrier()

    off = 0
    for sz in zsizes:
      pltpu.sync_copy(acc_sh.at[pl.ds(s * _ZR + off, sz)],
                      rows0.at[pl.ds(0, sz)])
      pltpu.sync_copy(rows0.at[pl.ds(0, sz)],
                      out_hbm.at[pl.ds(c * _N + s * _ZR + off, sz)])
      off += sz

  return seg


_seg_sum0 = _make_seg_sum(_W0)
_seg_sum = _make_seg_sum(_H)


def _bn_relu(t, g, b):
  mu = jnp.mean(t, axis=0, keepdims=True)
  d = t - mu
  var = jnp.mean(d * d, axis=0, keepdims=True)
  h = d * lax.rsqrt(var + _EPS) * g + b
  return jnp.maximum(h, 0.0)


def _tc_layer0(partials_ref, x_ref, wl_ref, bl_ref, wr_ref, g_ref, b_ref,
               wproj_ref, h_out, icnt_out):
  p = partials_ref[...]
  psum = p[0:_N, :] + p[_N:2 * _N, :]
  inv = 1.0 / jnp.maximum(psum[:, _H:_H + 1], 1.0)
  icnt_out[...] = inv
  mean = psum[:, 0:_H] * inv
  x = x_ref[0:_N, 0:_H]
  t = (jnp.dot(mean, wl_ref[...], preferred_element_type=jnp.float32)
       + bl_ref[...]
       + jnp.dot(x, wr_ref[...], preferred_element_type=jnp.float32))
  h = _bn_relu(t, g_ref[...], b_ref[...])
  h = h + jnp.dot(x, wproj_ref[...], preferred_element_type=jnp.float32)
  h_out[0:_N, :] = h
  h_out[_N:_NROWS, :] = jnp.zeros((_ZROWS, _H), jnp.float32)


def _tc_layer1(partials_ref, h_ref, icnt_ref, wl_ref, bl_ref, wr_ref, g_ref,
               b_ref, h_out):
  p = partials_ref[...]
  psum = p[0:_N, :] + p[_N:2 * _N, :]
  mean = psum * icnt_ref[...]
  h_in = h_ref[0:_N, :]
  t = (jnp.dot(mean, wl_ref[...], preferred_element_type=jnp.float32)
       + bl_ref[...]
       + jnp.dot(h_in, wr_ref[...], preferred_element_type=jnp.float32))
  h = _bn_relu(t, g_ref[...], b_ref[...]) + h_in
  h_out[0:_N, :] = h
  h_out[_N:_NROWS, :] = jnp.zeros((_ZROWS, _H), jnp.float32)


def _tc_layer2(partials_ref, h_ref, icnt_ref, wl_ref, bl_ref, wr_ref, g_ref,
               b_ref, wcls_ref, bcls_ref, out_ref):
  p = partials_ref[...]
  psum = p[0:_N, :] + p[_N:2 * _N, :]
  mean = psum * icnt_ref[...]
  h_in = h_ref[0:_N, :]
  t = (jnp.dot(mean, wl_ref[...], preferred_element_type=jnp.float32)
       + bl_ref[...]
       + jnp.dot(h_in, wr_ref[...], preferred_element_type=jnp.float32))
  h = _bn_relu(t, g_ref[...], b_ref[...]) + h_in
  out_ref[...] = (jnp.dot(h, wcls_ref[...], preferred_element_type=jnp.float32)
                  + bcls_ref[...])


def kernel(x, edge_index, W_l0, b_l0, W_r0, gamma0, beta0, W_l1, b_l1, W_r1,
           gamma1, beta1, W_l2, b_l2, W_r2, gamma2, beta2, W_proj, W_cls,
           b_cls):
  src = edge_index[0]
  dst = edge_index[1]
  npad = _EPAD - _E
  pad_ids = jnp.arange(npad, dtype=jnp.int32)
  src_pad = jnp.concatenate([src, _N + pad_ids % _ZROWS]).reshape(_IROWS, _K)
  dst_pad = jnp.concatenate([dst, pad_ids % _N]).reshape(_IROWS, _K)

  x_ext = jnp.concatenate(
      [x, jnp.ones((_N, 1), jnp.float32), jnp.zeros((_N, 15), jnp.float32)],
      axis=1)
  x_ext = jnp.concatenate(
      [x_ext, jnp.zeros((_ZROWS, _W0), jnp.float32)], axis=0)

  partials0 = _seg_sum0(x_ext, src_pad, dst_pad)
  h1, inv_cnt = pl.pallas_call(
      _tc_layer0,
      out_shape=(jax.ShapeDtypeStruct((_NROWS, _H), jnp.float32),
                 jax.ShapeDtypeStruct((_N, 1), jnp.float32)),
  )(partials0, x_ext, W_l0, b_l0, W_r0, gamma0, beta0, W_proj)

  partials1 = _seg_sum(h1, src_pad, dst_pad)
  h2 = pl.pallas_call(
      _tc_layer1, out_shape=jax.ShapeDtypeStruct((_NROWS, _H), jnp.float32),
  )(partials1, h1, inv_cnt, W_l1, b_l1, W_r1, gamma1, beta1)

  partials2 = _seg_sum(h2, src_pad, dst_pad)
  out = pl.pallas_call(
      _tc_layer2, out_shape=jax.ShapeDtypeStruct((_N, _C), jnp.float32),
  )(partials2, h2, inv_cnt, W_l2, b_l2, W_r2, gamma2, beta2, W_cls, b_cls)
  return out


# speedup vs baseline: 10.4709x; 1.0027x over previous
import functools

import jax
import jax.numpy as jnp
from jax import lax
from jax.experimental import pallas as pl
from jax.experimental.pallas import tpu as pltpu
from jax.experimental.pallas import tpu_sc as plsc

_N = 10000
_E = 320000
_H = 128
_C = 2
_EPS = 1e-5

_NC = 2
_NS = 16
_NW = _NC * _NS
_L = 16

_W0 = _H + 16
_K = 128
_G = 8
_BODY = 2 * _G
_NBODY = 5
_CPW = _BODY * _NBODY
_EPW = _CPW * _K
_EPAD = _NW * _EPW
_IROWS = _EPAD // _K
_ZR = _N // _NS
_ZROWS = 128
_NROWS = _N + _ZROWS

_mesh = plsc.VectorSubcoreMesh(core_axis_name="c", subcore_axis_name="s")


def _make_seg_sum(width):
  @functools.partial(
      pl.kernel,
      out_type=jax.ShapeDtypeStruct((_NC * _N, width), jnp.float32),
      mesh=_mesh,
      scratch_types=[
          pltpu.VMEM((_G, _K), jnp.int32),
          pltpu.VMEM((_G, _K), jnp.int32),
          pltpu.VMEM((_G, _K), jnp.int32),
          pltpu.VMEM((_G, _K), jnp.int32),
          pltpu.VMEM((_K, width), jnp.float32),
          pltpu.VMEM((_K, width), jnp.float32),
          pltpu.VMEM_SHARED((_N, width), jnp.float32),
          pltpu.SemaphoreType.DMA,
          pltpu.SemaphoreType.DMA,
          pltpu.SemaphoreType.DMA,
          pltpu.SemaphoreType.DMA,
          pltpu.SemaphoreType.DMA,
          pltpu.SemaphoreType.DMA,
      ],
      compiler_params=pltpu.CompilerParams(use_tc_tiling_on_sc=False),
  )
  def seg(h_hbm, src_hbm, dst_hbm, out_hbm, srca, dsta, srcb, dstb,
          rows0, rows1, acc_sh, semg0, semg1, sems0, sems1, semia, semib):
    c = lax.axis_index("c")
    s = lax.axis_index("s")
    w = s * _NC + c
    rows = (rows0, rows1)
    semg = (semg0, semg1)
    sems = (sems0, sems1)

    row0 = w * _CPW
    pltpu.async_copy(src_hbm.at[pl.ds(row0, _G)], srca, semia)
    pltpu.async_copy(dst_hbm.at[pl.ds(row0, _G)], dsta, semia)

    def zrow(i, carry):
      for j in range(width // _L):
        rows0[i, pl.ds(j * _L, _L)] = jnp.zeros((_L,), jnp.float32)
      return carry
    lax.fori_loop(0, _K, zrow, 0)
    zsizes = [_K] * (_ZR // _K) + ([_ZR % _K] if _ZR % _K else [])
    zdescs = []
    off = 0
    for sz in zsizes:
      zdescs.append(pltpu.async_copy(
          rows0.at[pl.ds(0, sz)], acc_sh.at[pl.ds(s * _ZR + off, sz)], sems0))
      off += sz
    for d in zdescs:
      d.wait()
    plsc.subcore_barrier()

    def body(m, carry):
      base = row0 + m * _BODY
      pltpu.make_async_copy(src_hbm.at[pl.ds(base, _G)], srca, semia).wait()
      pltpu.make_async_copy(dst_hbm.at[pl.ds(base, _G)], dsta, semia).wait()
      pltpu.async_copy(src_hbm.at[pl.ds(base + _G, _G)], srcb, semib)
      pltpu.async_copy(dst_hbm.at[pl.ds(base + _G, _G)], dstb, semib)

      descs_g = [None] * _BODY
      descs_s = [None] * _BODY
      descs_g[0] = pltpu.async_copy(h_hbm.at[srca.at[0]], rows0, semg0)
      for j in range(_BODY):
        sl = j % 2
        descs_g[j].wait()
        didx = dsta if j < _G else dstb
        descs_s[j] = pltpu.async_copy(
            rows[sl], acc_sh.at[didx.at[j % _G]], sems[sl], add=True)
        if j + 1 < _BODY:
          if j + 1 == _G:
            pltpu.make_async_copy(
                src_hbm.at[pl.ds(base + _G, _G)], srcb, semib).wait()
            pltpu.make_async_copy(
                dst_hbm.at[pl.ds(base + _G, _G)], dstb, semib).wait()
          if j >= 1:
            descs_s[j - 1].wait()
          nsrc = srca if j + 1 < _G else srcb
          descs_g[j + 1] = pltpu.async_copy(
              h_hbm.at[nsrc.at[(j + 1) % _G]], rows[1 - sl], semg[1 - sl])
        if j == _BODY - 1:
          nbase = lax.rem(base + _BODY, _IROWS)
          pltpu.async_copy(src_hbm.at[pl.ds(nbase, _G)], srca, semia)
          pltpu.async_copy(dst_hbm.at[pl.ds(nbase, _G)], dsta, semia)
      descs_s[_BODY - 2].wait()
      descs_s[_BODY - 1].wait()
      return carry

    lax.fori_loop(0, _NBODY, body, 0)
    pltpu.make_async_copy(src_hbm.at[pl.ds(row0, _G)], srca, semia).wait()
    pltpu.make_async_copy(dst_hbm.at[pl.ds(row0, _G)], dsta, semia).wait()
    plsc.subcore_barrier()

    odescs = []
    off = 0
    for sz in zsizes:
      odescs.append(pltpu.async_copy(
          acc_sh.at[pl.ds(s * _ZR + off, sz)],
          out_hbm.at[pl.ds(c * _N + s * _ZR + off, sz)], sems1))
      off += sz
    for d in odescs:
      d.wait()

  return seg


_seg_sum0 = _make_seg_sum(_W0)
_seg_sum = _make_seg_sum(_H)


def _bn_relu(t, g, b):
  mu = jnp.mean(t, axis=0, keepdims=True)
  d = t - mu
  var = jnp.mean(d * d, axis=0, keepdims=True)
  h = d * lax.rsqrt(var + _EPS) * g + b
  return jnp.maximum(h, 0.0)


def _tc_layer0(partials_ref, x_ref, wl_ref, bl_ref, wr_ref, g_ref, b_ref,
               wproj_ref, h_out, icnt_out):
  p = partials_ref[...]
  psum = p[0:_N, :] + p[_N:2 * _N, :]
  inv = 1.0 / jnp.maximum(psum[:, _H:_H + 1], 1.0)
  icnt_out[...] = inv
  mean = psum[:, 0:_H] * inv
  x = x_ref[0:_N, 0:_H]
  t = (jnp.dot(mean, wl_ref[...], preferred_element_type=jnp.float32)
       + bl_ref[...]
       + jnp.dot(x, wr_ref[...], preferred_element_type=jnp.float32))
  h = _bn_relu(t, g_ref[...], b_ref[...])
  h = h + jnp.dot(x, wproj_ref[...], preferred_element_type=jnp.float32)
  h_out[0:_N, :] = h
  h_out[_N:_NROWS, :] = jnp.zeros((_ZROWS, _H), jnp.float32)


def _tc_layer1(partials_ref, h_ref, icnt_ref, wl_ref, bl_ref, wr_ref, g_ref,
               b_ref, h_out):
  p = partials_ref[...]
  psum = p[0:_N, :] + p[_N:2 * _N, :]
  mean = psum * icnt_ref[...]
  h_in = h_ref[0:_N, :]
  t = (jnp.dot(mean, wl_ref[...], preferred_element_type=jnp.float32)
       + bl_ref[...]
       + jnp.dot(h_in, wr_ref[...], preferred_element_type=jnp.float32))
  h = _bn_relu(t, g_ref[...], b_ref[...]) + h_in
  h_out[0:_N, :] = h
  h_out[_N:_NROWS, :] = jnp.zeros((_ZROWS, _H), jnp.float32)


def _tc_layer2(partials_ref, h_ref, icnt_ref, wl_ref, bl_ref, wr_ref, g_ref,
               b_ref, wcls_ref, bcls_ref, out_ref):
  p = partials_ref[...]
  psum = p[0:_N, :] + p[_N:2 * _N, :]
  mean = psum * icnt_ref[...]
  h_in = h_ref[0:_N, :]
  t = (jnp.dot(mean, wl_ref[...], preferred_element_type=jnp.float32)
       + bl_ref[...]
       + jnp.dot(h_in, wr_ref[...], preferred_element_type=jnp.float32))
  h = _bn_relu(t, g_ref[...], b_ref[...]) + h_in
  out_ref[...] = (jnp.dot(h, wcls_ref[...], preferred_element_type=jnp.float32)
                  + bcls_ref[...])


def kernel(x, edge_index, W_l0, b_l0, W_r0, gamma0, beta0, W_l1, b_l1, W_r1,
           gamma1, beta1, W_l2, b_l2, W_r2, gamma2, beta2, W_proj, W_cls,
           b_cls):
  src = edge_index[0]
  dst = edge_index[1]
  npad = _EPAD - _E
  pad_ids = jnp.arange(npad, dtype=jnp.int32)
  src_pad = jnp.concatenate([src, _N + pad_ids % _ZROWS]).reshape(_IROWS, _K)
  dst_pad = jnp.concatenate([dst, pad_ids % _N]).reshape(_IROWS, _K)

  x_ext = jnp.concatenate(
      [x, jnp.ones((_N, 1), jnp.float32), jnp.zeros((_N, 15), jnp.float32)],
      axis=1)
  x_ext = jnp.concatenate(
      [x_ext, jnp.zeros((_ZROWS, _W0), jnp.float32)], axis=0)

  partials0 = _seg_sum0(x_ext, src_pad, dst_pad)
  h1, inv_cnt = pl.pallas_call(
      _tc_layer0,
      out_shape=(jax.ShapeDtypeStruct((_NROWS, _H), jnp.float32),
                 jax.ShapeDtypeStruct((_N, 1), jnp.float32)),
  )(partials0, x_ext, W_l0, b_l0, W_r0, gamma0, beta0, W_proj)

  partials1 = _seg_sum(h1, src_pad, dst_pad)
  h2 = pl.pallas_call(
      _tc_layer1, out_shape=jax.ShapeDtypeStruct((_NROWS, _H), jnp.float32),
  )(partials1, h1, inv_cnt, W_l1, b_l1, W_r1, gamma1, beta1)

  partials2 = _seg_sum(h2, src_pad, dst_pad)
  out = pl.pallas_call(
      _tc_layer2, out_shape=jax.ShapeDtypeStruct((_N, _C), jnp.float32),
  )(partials2, h2, inv_cnt, W_l2, b_l2, W_r2, gamma2, beta2, W_cls, b_cls)
  return out


# speedup vs baseline: 11.2168x; 1.0712x over previous
import functools

import jax
import jax.numpy as jnp
from jax import lax
from jax.experimental import pallas as pl
from jax.experimental.pallas import tpu as pltpu
from jax.experimental.pallas import tpu_sc as plsc

_N = 10000
_E = 320000
_H = 128
_C = 2
_EPS = 1e-5

_NC = 2
_NS = 16
_NW = _NC * _NS
_L = 16

_W0 = _H + 16
_K = 128
_G = 8
_BODY = 2 * _G
_NBODY = 5
_CPW = _BODY * _NBODY
_EPW = _CPW * _K
_EPAD = _NW * _EPW
_IROWS = _EPAD // _K
_ZR = _N // _NS
_ZROWS = 128
_NROWS = _N + _ZROWS

_mesh = plsc.VectorSubcoreMesh(core_axis_name="c", subcore_axis_name="s")


def _make_seg_sum(width, counts=False, g=_G):
  body_n = 2 * g
  nbody = _CPW // body_n
  out_type = jax.ShapeDtypeStruct((_NC * _N, width), jnp.float32)
  scratch = [
      pltpu.VMEM((g, _K), jnp.int32),
      pltpu.VMEM((g, _K), jnp.int32),
      pltpu.VMEM((g, _K), jnp.int32),
      pltpu.VMEM((g, _K), jnp.int32),
      pltpu.VMEM((_K, width), jnp.float32),
      pltpu.VMEM((_K, width), jnp.float32),
      pltpu.VMEM_SHARED((_N, width), jnp.float32),
      pltpu.SemaphoreType.DMA,
      pltpu.SemaphoreType.DMA,
      pltpu.SemaphoreType.DMA,
      pltpu.SemaphoreType.DMA,
      pltpu.SemaphoreType.DMA,
      pltpu.SemaphoreType.DMA,
  ]
  if counts:
    out_type = (out_type, jax.ShapeDtypeStruct((_NC * _N, 16), jnp.float32))
    scratch += [
        pltpu.VMEM((_K, 16), jnp.float32),
        pltpu.VMEM((_K, 16), jnp.float32),
        pltpu.VMEM_SHARED((_N, 16), jnp.float32),
        pltpu.SemaphoreType.DMA,
    ]

  @functools.partial(
      pl.kernel,
      out_type=out_type,
      mesh=_mesh,
      scratch_types=scratch,
      compiler_params=pltpu.CompilerParams(use_tc_tiling_on_sc=False),
  )
  def seg(h_hbm, src_hbm, dst_hbm, out_hbm, *rest):
    if counts:
      (cnt_hbm, srca, dsta, srcb, dstb, rows0, rows1, acc_sh, semg0, semg1,
       sems0, sems1, semia, semib, ones_v, zc_v, cnt_sh, semc) = rest
    else:
      (srca, dsta, srcb, dstb, rows0, rows1, acc_sh, semg0, semg1,
       sems0, sems1, semia, semib) = rest
    c = lax.axis_index("c")
    s = lax.axis_index("s")
    w = s * _NC + c
    rows = (rows0, rows1)
    semg = (semg0, semg1)
    sems = (sems0, sems1)

    row0 = w * _CPW
    pltpu.async_copy(src_hbm.at[pl.ds(row0, g)], srca, semia)
    pltpu.async_copy(dst_hbm.at[pl.ds(row0, g)], dsta, semia)

    def zrow(i, carry):
      for j in range(width // _L):
        rows0[i, pl.ds(j * _L, _L)] = jnp.zeros((_L,), jnp.float32)
      return carry
    lax.fori_loop(0, _K, zrow, 0)
    if counts:
      def orow(i, carry):
        ones_v[i, pl.ds(0, _L)] = jnp.ones((_L,), jnp.float32)
        zc_v[i, pl.ds(0, _L)] = jnp.zeros((_L,), jnp.float32)
        return carry
      lax.fori_loop(0, _K, orow, 0)
    zsizes = [_K] * (_ZR // _K) + ([_ZR % _K] if _ZR % _K else [])
    zdescs = []
    off = 0
    for sz in zsizes:
      zdescs.append(pltpu.async_copy(
          rows0.at[pl.ds(0, sz)], acc_sh.at[pl.ds(s * _ZR + off, sz)], sems0))
      if counts:
        zdescs.append(pltpu.async_copy(
            zc_v.at[pl.ds(0, sz)], cnt_sh.at[pl.ds(s * _ZR + off, sz)], sems0))
      off += sz
    for d in zdescs:
      d.wait()
    plsc.subcore_barrier()

    def body(m, carry):
      base = row0 + m * body_n
      pltpu.make_async_copy(src_hbm.at[pl.ds(base, g)], srca, semia).wait()
      pltpu.make_async_copy(dst_hbm.at[pl.ds(base, g)], dsta, semia).wait()
      pltpu.async_copy(src_hbm.at[pl.ds(base + g, g)], srcb, semib)
      pltpu.async_copy(dst_hbm.at[pl.ds(base + g, g)], dstb, semib)

      descs_g = [None] * body_n
      descs_s = [None] * body_n
      descs_c = [None] * body_n
      descs_g[0] = pltpu.async_copy(h_hbm.at[srca.at[0]], rows0, semg0)
      for j in range(body_n):
        sl = j % 2
        descs_g[j].wait()
        didx = dsta if j < g else dstb
        descs_s[j] = pltpu.async_copy(
            rows[sl], acc_sh.at[didx.at[j % g]], sems[sl], add=True)
        if counts:
          descs_c[j] = pltpu.async_copy(
              ones_v, cnt_sh.at[didx.at[j % g]], semc, add=True)
        if j + 1 < body_n:
          if j + 1 == g:
            pltpu.make_async_copy(
                src_hbm.at[pl.ds(base + g, g)], srcb, semib).wait()
            pltpu.make_async_copy(
                dst_hbm.at[pl.ds(base + g, g)], dstb, semib).wait()
          if j >= 1:
            descs_s[j - 1].wait()
          nsrc = srca if j + 1 < g else srcb
          descs_g[j + 1] = pltpu.async_copy(
              h_hbm.at[nsrc.at[(j + 1) % g]], rows[1 - sl], semg[1 - sl])
        if j == body_n - 1:
          nbase = lax.rem(base + body_n, _IROWS)
          pltpu.async_copy(src_hbm.at[pl.ds(nbase, g)], srca, semia)
          pltpu.async_copy(dst_hbm.at[pl.ds(nbase, g)], dsta, semia)
      descs_s[body_n - 2].wait()
      descs_s[body_n - 1].wait()
      if counts:
        for j in range(body_n):
          descs_c[j].wait()
      return carry

    lax.fori_loop(0, nbody, body, 0)
    pltpu.make_async_copy(src_hbm.at[pl.ds(row0, g)], srca, semia).wait()
    pltpu.make_async_copy(dst_hbm.at[pl.ds(row0, g)], dsta, semia).wait()
    plsc.subcore_barrier()

    odescs = []
    off = 0
    for sz in zsizes:
      odescs.append(pltpu.async_copy(
          acc_sh.at[pl.ds(s * _ZR + off, sz)],
          out_hbm.at[pl.ds(c * _N + s * _ZR + off, sz)], sems1))
      if counts:
        odescs.append(pltpu.async_copy(
            cnt_sh.at[pl.ds(s * _ZR + off, sz)],
            cnt_hbm.at[pl.ds(c * _N + s * _ZR + off, sz)], sems1))
      off += sz
    for d in odescs:
      d.wait()

  return seg


_seg_sum0 = _make_seg_sum(_H, counts=True, g=4)
_seg_sum = _make_seg_sum(_H)


def _bn_relu(t, g, b):
  mu = jnp.mean(t, axis=0, keepdims=True)
  d = t - mu
  var = jnp.mean(d * d, axis=0, keepdims=True)
  h = d * lax.rsqrt(var + _EPS) * g + b
  return jnp.maximum(h, 0.0)


def _tc_layer0(partials_ref, cnt_ref, x_ref, wl_ref, bl_ref, wr_ref, g_ref,
               b_ref, wproj_ref, h_out, icnt_out):
  p = partials_ref[...]
  psum = p[0:_N, :] + p[_N:2 * _N, :]
  cnt = cnt_ref[0:_N, 0:1] + cnt_ref[_N:2 * _N, 0:1]
  rowid = lax.broadcasted_iota(jnp.int32, (_N, 1), 0)
  cnt = cnt - jnp.where(rowid < _EPAD - _E, 1.0, 0.0)
  inv = 1.0 / jnp.maximum(cnt, 1.0)
  icnt_out[...] = inv
  mean = psum * inv
  x = x_ref[0:_N, 0:_H]
  t = (jnp.dot(mean, wl_ref[...], preferred_element_type=jnp.float32)
       + bl_ref[...]
       + jnp.dot(x, wr_ref[...], preferred_element_type=jnp.float32))
  h = _bn_relu(t, g_ref[...], b_ref[...])
  h = h + jnp.dot(x, wproj_ref[...], preferred_element_type=jnp.float32)
  h_out[0:_N, :] = h
  h_out[_N:_NROWS, :] = jnp.zeros((_ZROWS, _H), jnp.float32)


def _tc_layer1(partials_ref, h_ref, icnt_ref, wl_ref, bl_ref, wr_ref, g_ref,
               b_ref, h_out):
  p = partials_ref[...]
  psum = p[0:_N, :] + p[_N:2 * _N, :]
  mean = psum * icnt_ref[...]
  h_in = h_ref[0:_N, :]
  t = (jnp.dot(mean, wl_ref[...], preferred_element_type=jnp.float32)
       + bl_ref[...]
       + jnp.dot(h_in, wr_ref[...], preferred_element_type=jnp.float32))
  h = _bn_relu(t, g_ref[...], b_ref[...]) + h_in
  h_out[0:_N, :] = h
  h_out[_N:_NROWS, :] = jnp.zeros((_ZROWS, _H), jnp.float32)


def _tc_layer2(partials_ref, h_ref, icnt_ref, wl_ref, bl_ref, wr_ref, g_ref,
               b_ref, wcls_ref, bcls_ref, out_ref):
  p = partials_ref[...]
  psum = p[0:_N, :] + p[_N:2 * _N, :]
  mean = psum * icnt_ref[...]
  h_in = h_ref[0:_N, :]
  t = (jnp.dot(mean, wl_ref[...], preferred_element_type=jnp.float32)
       + bl_ref[...]
       + jnp.dot(h_in, wr_ref[...], preferred_element_type=jnp.float32))
  h = _bn_relu(t, g_ref[...], b_ref[...]) + h_in
  out_ref[...] = (jnp.dot(h, wcls_ref[...], preferred_element_type=jnp.float32)
                  + bcls_ref[...])


def kernel(x, edge_index, W_l0, b_l0, W_r0, gamma0, beta0, W_l1, b_l1, W_r1,
           gamma1, beta1, W_l2, b_l2, W_r2, gamma2, beta2, W_proj, W_cls,
           b_cls):
  src = edge_index[0]
  dst = edge_index[1]
  npad = _EPAD - _E
  pad_ids = jnp.arange(npad, dtype=jnp.int32)
  src_pad = jnp.concatenate([src, _N + pad_ids % _ZROWS]).reshape(_IROWS, _K)
  dst_pad = jnp.concatenate([dst, pad_ids % _N]).reshape(_IROWS, _K)

  x_pad = jnp.concatenate([x, jnp.zeros((_ZROWS, _H), jnp.float32)], axis=0)

  partials0, cnt = _seg_sum0(x_pad, src_pad, dst_pad)
  h1, inv_cnt = pl.pallas_call(
      _tc_layer0,
      out_shape=(jax.ShapeDtypeStruct((_NROWS, _H), jnp.float32),
                 jax.ShapeDtypeStruct((_N, 1), jnp.float32)),
  )(partials0, cnt, x_pad, W_l0, b_l0, W_r0, gamma0, beta0, W_proj)

  partials1 = _seg_sum(h1, src_pad, dst_pad)
  h2 = pl.pallas_call(
      _tc_layer1, out_shape=jax.ShapeDtypeStruct((_NROWS, _H), jnp.float32),
  )(partials1, h1, inv_cnt, W_l1, b_l1, W_r1, gamma1, beta1)

  partials2 = _seg_sum(h2, src_pad, dst_pad)
  out = pl.pallas_call(
      _tc_layer2, out_shape=jax.ShapeDtypeStruct((_N, _C), jnp.float32),
  )(partials2, h2, inv_cnt, W_l2, b_l2, W_r2, gamma2, beta2, W_cls, b_cls)
  return out


# speedup vs baseline: 11.2312x; 1.0013x over previous
import functools

import jax
import jax.numpy as jnp
from jax import lax
from jax.experimental import pallas as pl
from jax.experimental.pallas import tpu as pltpu
from jax.experimental.pallas import tpu_sc as plsc

_N = 10000
_E = 320000
_H = 128
_C = 2
_EPS = 1e-5

_NC = 2
_NS = 16
_NW = _NC * _NS
_L = 16

_K = 128
_G = 8
_BODY = 2 * _G
_NBODY = 5
_CPW = _BODY * _NBODY
_EPW = _CPW * _K
_EPAD = _NW * _EPW
_IROWS = _EPAD // _K
_ZR = _N // _NS
_ZROWS = 128
_NROWS = _N + _ZROWS

_mesh = plsc.VectorSubcoreMesh(core_axis_name="c", subcore_axis_name="s")


def _make_seg_sum(width, counts=False, g=_G):
  body_n = 2 * g
  nbody = _CPW // body_n
  out_type = jax.ShapeDtypeStruct((_NC * _N, width), jnp.float32)
  scratch = [
      pltpu.VMEM((g, _K), jnp.int32),
      pltpu.VMEM((g, _K), jnp.int32),
      pltpu.VMEM((g, _K), jnp.int32),
      pltpu.VMEM((g, _K), jnp.int32),
      pltpu.VMEM((_K, width), jnp.float32),
      pltpu.VMEM((_K, width), jnp.float32),
      pltpu.VMEM_SHARED((_N, width), jnp.float32),
      pltpu.SemaphoreType.DMA,
      pltpu.SemaphoreType.DMA,
      pltpu.SemaphoreType.DMA,
      pltpu.SemaphoreType.DMA,
      pltpu.SemaphoreType.DMA,
      pltpu.SemaphoreType.DMA,
  ]
  if counts:
    out_type = (out_type, jax.ShapeDtypeStruct((_NC * _N, 16), jnp.float32))
    scratch += [
        pltpu.VMEM((_K, 16), jnp.float32),
        pltpu.VMEM((_K, 16), jnp.float32),
        pltpu.VMEM_SHARED((_N, 16), jnp.float32),
        pltpu.SemaphoreType.DMA,
    ]

  @functools.partial(
      pl.kernel,
      out_type=out_type,
      mesh=_mesh,
      scratch_types=scratch,
      compiler_params=pltpu.CompilerParams(use_tc_tiling_on_sc=False),
  )
  def seg(h_hbm, src_hbm, dst_hbm, out_hbm, *rest):
    if counts:
      (cnt_hbm, srca, dsta, srcb, dstb, rows0, rows1, acc_sh, semg0, semg1,
       sems0, sems1, semia, semib, ones_v, zc_v, cnt_sh, semc) = rest
    else:
      (srca, dsta, srcb, dstb, rows0, rows1, acc_sh, semg0, semg1,
       sems0, sems1, semia, semib) = rest
    c = lax.axis_index("c")
    s = lax.axis_index("s")
    w = s * _NC + c
    rows = (rows0, rows1)
    semg = (semg0, semg1)
    sems = (sems0, sems1)

    row0 = w * _CPW
    pltpu.async_copy(src_hbm.at[pl.ds(row0, g)], srca, semia)
    pltpu.async_copy(dst_hbm.at[pl.ds(row0, g)], dsta, semia)

    def zrow(i, carry):
      for j in range(width // _L):
        rows0[i, pl.ds(j * _L, _L)] = jnp.zeros((_L,), jnp.float32)
      return carry
    lax.fori_loop(0, _K, zrow, 0)
    if counts:
      def orow(i, carry):
        ones_v[i, pl.ds(0, _L)] = jnp.ones((_L,), jnp.float32)
        zc_v[i, pl.ds(0, _L)] = jnp.zeros((_L,), jnp.float32)
        return carry
      lax.fori_loop(0, _K, orow, 0)
    zsizes = [_K] * (_ZR // _K) + ([_ZR % _K] if _ZR % _K else [])
    zdescs = []
    off = 0
    for sz in zsizes:
      zdescs.append(pltpu.async_copy(
          rows0.at[pl.ds(0, sz)], acc_sh.at[pl.ds(s * _ZR + off, sz)], sems0))
      if counts:
        zdescs.append(pltpu.async_copy(
            zc_v.at[pl.ds(0, sz)], cnt_sh.at[pl.ds(s * _ZR + off, sz)], sems0))
      off += sz
    for d in zdescs:
      d.wait()
    plsc.subcore_barrier()

    def body(m, carry):
      base = row0 + m * body_n
      pltpu.make_async_copy(src_hbm.at[pl.ds(base, g)], srca, semia).wait()
      pltpu.make_async_copy(dst_hbm.at[pl.ds(base, g)], dsta, semia).wait()
      pltpu.async_copy(src_hbm.at[pl.ds(base + g, g)], srcb, semib)
      pltpu.async_copy(dst_hbm.at[pl.ds(base + g, g)], dstb, semib)

      descs_g = [None] * body_n
      descs_s = [None] * body_n
      descs_c = [None] * body_n
      descs_g[0] = pltpu.async_copy(h_hbm.at[srca.at[0]], rows0, semg0)
      for j in range(body_n):
        sl = j % 2
        descs_g[j].wait()
        didx = dsta if j < g else dstb
        descs_s[j] = pltpu.async_copy(
            rows[sl], acc_sh.at[didx.at[j % g]], sems[sl], add=True)
        if counts:
          descs_c[j] = pltpu.async_copy(
              ones_v, cnt_sh.at[didx.at[j % g]], semc, add=True)
        if j + 1 < body_n:
          if j + 1 == g:
            pltpu.make_async_copy(
                src_hbm.at[pl.ds(base + g, g)], srcb, semib).wait()
            pltpu.make_async_copy(
                dst_hbm.at[pl.ds(base + g, g)], dstb, semib).wait()
          if j >= 1:
            descs_s[j - 1].wait()
          nsrc = srca if j + 1 < g else srcb
          descs_g[j + 1] = pltpu.async_copy(
              h_hbm.at[nsrc.at[(j + 1) % g]], rows[1 - sl], semg[1 - sl])
        if j == body_n - 1:
          nbase = lax.rem(base + body_n, _IROWS)
          pltpu.async_copy(src_hbm.at[pl.ds(nbase, g)], srca, semia)
          pltpu.async_copy(dst_hbm.at[pl.ds(nbase, g)], dsta, semia)
      descs_s[body_n - 2].wait()
      descs_s[body_n - 1].wait()
      if counts:
        for j in range(body_n):
          descs_c[j].wait()
      return carry

    lax.fori_loop(0, nbody, body, 0)
    pltpu.make_async_copy(src_hbm.at[pl.ds(row0, g)], srca, semia).wait()
    pltpu.make_async_copy(dst_hbm.at[pl.ds(row0, g)], dsta, semia).wait()
    plsc.subcore_barrier()

    odescs = []
    off = 0
    for sz in zsizes:
      odescs.append(pltpu.async_copy(
          acc_sh.at[pl.ds(s * _ZR + off, sz)],
          out_hbm.at[pl.ds(c * _N + s * _ZR + off, sz)], sems1))
      if counts:
        odescs.append(pltpu.async_copy(
            cnt_sh.at[pl.ds(s * _ZR + off, sz)],
            cnt_hbm.at[pl.ds(c * _N + s * _ZR + off, sz)], sems1))
      off += sz
    for d in odescs:
      d.wait()

  return seg


_seg_sum0 = _make_seg_sum(_H, counts=True, g=4)
_seg_sum = _make_seg_sum(_H)


def _bn_relu(t, g, b):
  mu = jnp.mean(t, axis=0, keepdims=True)
  d = t - mu
  var = jnp.mean(d * d, axis=0, keepdims=True)
  h = d * lax.rsqrt(var + _EPS) * g + b
  return jnp.maximum(h, 0.0)


def _tc_layer0(partials_ref, cnt_ref, x_ref, wl_ref, bl_ref, wr_ref, g_ref,
               b_ref, wproj_ref, h_out, icnt_out):
  p = partials_ref[...]
  psum = p[0:_N, :] + p[_N:2 * _N, :]
  cnt = cnt_ref[0:_N, 0:1] + cnt_ref[_N:2 * _N, 0:1]
  rowid = lax.broadcasted_iota(jnp.int32, (_N, 1), 0)
  cnt = cnt - jnp.where(rowid < _EPAD - _E, 1.0, 0.0)
  inv = 1.0 / jnp.maximum(cnt, 1.0)
  icnt_out[...] = inv
  mean = psum * inv
  x = x_ref[0:_N, 0:_H]
  t = (jnp.dot(mean, wl_ref[...], preferred_element_type=jnp.float32)
       + bl_ref[...]
       + jnp.dot(x, wr_ref[...], preferred_element_type=jnp.float32))
  h = _bn_relu(t, g_ref[...], b_ref[...])
  h = h + jnp.dot(x, wproj_ref[...], preferred_element_type=jnp.float32)
  h_out[0:_N, :] = h
  h_out[_N:_NROWS, :] = jnp.zeros((_ZROWS, _H), jnp.float32)


def _tc_layer1(partials_ref, h_ref, icnt_ref, wl_ref, bl_ref, wr_ref, g_ref,
               b_ref, h_out):
  p = partials_ref[...]
  psum = p[0:_N, :] + p[_N:2 * _N, :]
  mean = psum * icnt_ref[...]
  h_in = h_ref[0:_N, :]
  t = (jnp.dot(mean, wl_ref[...], preferred_element_type=jnp.float32)
       + bl_ref[...]
       + jnp.dot(h_in, wr_ref[...], preferred_element_type=jnp.float32))
  h = _bn_relu(t, g_ref[...], b_ref[...]) + h_in
  h_out[0:_N, :] = h
  h_out[_N:_NROWS, :] = jnp.zeros((_ZROWS, _H), jnp.float32)


def _tc_layer2(partials_ref, h_ref, icnt_ref, wl_ref, bl_ref, wr_ref, g_ref,
               b_ref, wcls_ref, bcls_ref, out_ref):
  p = partials_ref[...]
  psum = p[0:_N, :] + p[_N:2 * _N, :]
  mean = psum * icnt_ref[...]
  h_in = h_ref[0:_N, :]
  t = (jnp.dot(mean, wl_ref[...], preferred_element_type=jnp.float32)
       + bl_ref[...]
       + jnp.dot(h_in, wr_ref[...], preferred_element_type=jnp.float32))
  h = _bn_relu(t, g_ref[...], b_ref[...]) + h_in
  out_ref[...] = (jnp.dot(h, wcls_ref[...], preferred_element_type=jnp.float32)
                  + bcls_ref[...])


def kernel(x, edge_index, W_l0, b_l0, W_r0, gamma0, beta0, W_l1, b_l1, W_r1,
           gamma1, beta1, W_l2, b_l2, W_r2, gamma2, beta2, W_proj, W_cls,
           b_cls):
  src = edge_index[0]
  dst = edge_index[1]
  npad = _EPAD - _E
  pad_ids = jnp.arange(npad, dtype=jnp.int32)
  src_pad = jnp.concatenate([src, _N + pad_ids % _ZROWS]).reshape(_IROWS, _K)
  dst_pad = jnp.concatenate([dst, pad_ids % _N]).reshape(_IROWS, _K)

  x_pad = jnp.concatenate([x, jnp.zeros((_ZROWS, _H), jnp.float32)], axis=0)

  partials0, cnt = _seg_sum0(x_pad, src_pad, dst_pad)
  h1, inv_cnt = pl.pallas_call(
      _tc_layer0,
      out_shape=(jax.ShapeDtypeStruct((_NROWS, _H), jnp.float32),
                 jax.ShapeDtypeStruct((_N, 1), jnp.float32)),
  )(partials0, cnt, x_pad, W_l0, b_l0, W_r0, gamma0, beta0, W_proj)

  partials1 = _seg_sum(h1, src_pad, dst_pad)
  h2 = pl.pallas_call(
      _tc_layer1, out_shape=jax.ShapeDtypeStruct((_NROWS, _H), jnp.float32),
  )(partials1, h1, inv_cnt, W_l1, b_l1, W_r1, gamma1, beta1)

  partials2 = _seg_sum(h2, src_pad, dst_pad)
  out = pl.pallas_call(
      _tc_layer2, out_shape=jax.ShapeDtypeStruct((_N, _C), jnp.float32),
  )(partials2, h2, inv_cnt, W_l2, b_l2, W_r2, gamma2, beta2, W_cls, b_cls)
  return out
